# baseline (speedup 1.0000x reference)
import functools

import jax
import jax.numpy as jnp
from jax import lax
from jax.experimental import pallas as pl
from jax.experimental.pallas import tpu as pltpu
from jax.experimental.pallas import tpu_sc as plsc

B = 4
S = 2048
D = 768
L = 16
NC = 2
NS = 16
NW = NC * NS
CHUNK = S // NW
QROWS = 8
NQ = CHUNK // QROWS


def _emb_body(idx_hbm, emb_hbm, pos_hbm, sos_hbm, out_hbm,
              idx_v, rows0, rows1, pos0, pos1, sos_v,
              gsem0, gsem1, ssem0, ssem1):
    wid = lax.axis_index("s") * NC + lax.axis_index("c")
    base = wid * CHUNK
    rows = (rows0, rows1)
    posb = (pos0, pos1)
    gsem = (gsem0, gsem1)
    ssem = (ssem0, ssem1)
    W = B * QROWS

    pltpu.sync_copy(idx_hbm.at[pl.ds(wid * B * CHUNK, B * CHUNK)], idx_v)

    @pl.when(wid == 0)
    def _load_sos():
        pltpu.sync_copy(sos_hbm, sos_v)

    def _start_gathers(q):
        p = q % 2
        return [
            pltpu.async_copy(
                pos_hbm.at[pl.ds(base + q * QROWS, QROWS), :], posb[p],
                gsem[p]),
            pltpu.async_copy(
                emb_hbm.at[idx_v.at[pl.ds(q * W, W)]], rows[p], gsem[p]),
        ]

    def _run_add(p):
        @plsc.parallel_loop(0, QROWS, 1, unroll=1)
        def _add_pos_row(r):
            for j in range(D // L):
                sl = pl.ds(j * L, L)
                pv = posb[p][r, sl]
                for b in range(B):
                    plsc.addupdate(rows[p].at[b * QROWS + r, sl], pv)

    g = _start_gathers(0)
    store_pending = [None, None]
    for q in range(NQ):
        p = q % 2
        if q + 1 < NQ:
            np_ = (q + 1) % 2
            if store_pending[np_] is not None:
                for sp in store_pending[np_]:
                    sp.wait()
                store_pending[np_] = None
            g_next = _start_gathers(q + 1)
        for d in g:
            d.wait()

        _run_add(p)

        if q == 0:
            @pl.when(wid == 0)
            def _fix_sos_row():
                for j in range(D // L):
                    sl = pl.ds(j * L, L)
                    v = sos_v[sl] + posb[0][0, sl]
                    for b in range(B):
                        rows[0][b * QROWS, sl] = v

        store_pending[p] = [pltpu.async_copy(
            rows[p].at[pl.ds(b * QROWS, QROWS)],
            out_hbm.at[b, pl.ds(base + q * QROWS, QROWS), :],
            ssem[p]) for b in range(B)]
        if q + 1 < NQ:
            g = g_next
    for pend in store_pending:
        if pend is not None:
            for sp in pend:
                sp.wait()


@functools.partial(jax.jit, static_argnames=())
def _run(idx_flat, emb_table, pos_table, sos_token):
    mesh = plsc.VectorSubcoreMesh(core_axis_name="c", subcore_axis_name="s")
    f = pl.kernel(
        _emb_body,
        out_type=jax.ShapeDtypeStruct((B, S, D), jnp.float32),
        mesh=mesh,
        scratch_types=(
            [pltpu.VMEM((B * CHUNK,), jnp.int32)]
            + [pltpu.VMEM((B * QROWS, D), jnp.float32) for _ in range(2)]
            + [pltpu.VMEM((QROWS, D), jnp.float32) for _ in range(2)]
            + [pltpu.VMEM((D,), jnp.float32)]
            + [pltpu.SemaphoreType.DMA for _ in range(4)]
        ),
    )
    return f(idx_flat, emb_table, pos_table, sos_token)


def kernel(x, emb_table, pos_table, sos_token):
    idx = jnp.concatenate(
        [jnp.zeros((B, 1), jnp.int32), x[:, :-1].astype(jnp.int32)], axis=1
    )
    idx = idx.reshape(B, NW, NQ, QROWS).transpose(1, 2, 0, 3).reshape(-1)
    return _run(idx, emb_table, pos_table, sos_token)


# speedup vs baseline: 1.0476x; 1.0476x over previous
import functools

import jax
import jax.numpy as jnp
from jax import lax
from jax.experimental import pallas as pl
from jax.experimental.pallas import tpu as pltpu
from jax.experimental.pallas import tpu_sc as plsc

B = 4
S = 2048
D = 768
L = 16
NC = 2
NS = 16
NW = NC * NS
CHUNK = S // NW
QROWS = 16
NQ = CHUNK // QROWS


def _emb_body(idx_hbm, emb_hbm, pos_hbm, sos_hbm, out_hbm,
              idx_v, rows0, rows1, pos0, pos1, sos_v,
              gsem0, gsem1, ssem0, ssem1):
    wid = lax.axis_index("s") * NC + lax.axis_index("c")
    base = wid * CHUNK
    rows = (rows0, rows1)
    posb = (pos0, pos1)
    gsem = (gsem0, gsem1)
    ssem = (ssem0, ssem1)
    W = B * QROWS

    pltpu.sync_copy(idx_hbm.at[pl.ds(wid * B * CHUNK, B * CHUNK)], idx_v)

    @pl.when(wid == 0)
    def _load_sos():
        pltpu.sync_copy(sos_hbm, sos_v)

    def _start_gathers(q):
        p = q % 2
        return [
            pltpu.async_copy(
                pos_hbm.at[pl.ds(base + q * QROWS, QROWS), :], posb[p],
                gsem[p]),
            pltpu.async_copy(
                emb_hbm.at[idx_v.at[pl.ds(q * W, W)]], rows[p], gsem[p]),
        ]

    def _run_add(p):
        @plsc.parallel_loop(0, QROWS, 1, unroll=1)
        def _add_pos_row(r):
            for j in range(D // L):
                sl = pl.ds(j * L, L)
                pv = posb[p][r, sl]
                for b in range(B):
                    plsc.addupdate(rows[p].at[b * QROWS + r, sl], pv)

    g = _start_gathers(0)
    store_pending = [None, None]
    for q in range(NQ):
        p = q % 2
        if q + 1 < NQ:
            np_ = (q + 1) % 2
            if store_pending[np_] is not None:
                for sp in store_pending[np_]:
                    sp.wait()
                store_pending[np_] = None
            g_next = _start_gathers(q + 1)
        for d in g:
            d.wait()

        _run_add(p)

        if q == 0:
            @pl.when(wid == 0)
            def _fix_sos_row():
                for j in range(D // L):
                    sl = pl.ds(j * L, L)
                    v = sos_v[sl] + posb[0][0, sl]
                    for b in range(B):
                        rows[0][b * QROWS, sl] = v

        store_pending[p] = [pltpu.async_copy(
            rows[p].at[pl.ds(b * QROWS, QROWS)],
            out_hbm.at[b, pl.ds(base + q * QROWS, QROWS), :],
            ssem[p]) for b in range(B)]
        if q + 1 < NQ:
            g = g_next
    for pend in store_pending:
        if pend is not None:
            for sp in pend:
                sp.wait()


@functools.partial(jax.jit, static_argnames=())
def _run(idx_flat, emb_table, pos_table, sos_token):
    mesh = plsc.VectorSubcoreMesh(core_axis_name="c", subcore_axis_name="s")
    f = pl.kernel(
        _emb_body,
        out_type=jax.ShapeDtypeStruct((B, S, D), jnp.float32),
        mesh=mesh,
        scratch_types=(
            [pltpu.VMEM((B * CHUNK,), jnp.int32)]
            + [pltpu.VMEM((B * QROWS, D), jnp.float32) for _ in range(2)]
            + [pltpu.VMEM((QROWS, D), jnp.float32) for _ in range(2)]
            + [pltpu.VMEM((D,), jnp.float32)]
            + [pltpu.SemaphoreType.DMA for _ in range(4)]
        ),
    )
    return f(idx_flat, emb_table, pos_table, sos_token)


def kernel(x, emb_table, pos_table, sos_token):
    idx = jnp.concatenate(
        [jnp.zeros((B, 1), jnp.int32), x[:, :-1].astype(jnp.int32)], axis=1
    )
    idx = idx.reshape(B, NW, NQ, QROWS).transpose(1, 2, 0, 3).reshape(-1)
    return _run(idx, emb_table, pos_table, sos_token)
